# baseline (speedup 1.0000x reference)
import jax
import jax.numpy as jnp
from jax import lax
from jax.experimental import pallas as pl

_EPS = 1e-10


def _body(cand_ref, beh_ref, rid_ref, w1_ref, b1_ref, alpha_ref, w2_ref,
          b2_ref, out_ref):
    i = pl.program_id(0)
    TB, D = beh_ref.shape
    B = cand_ref.shape[0]
    H = w1_ref.shape[1]
    DD = D * D
    f32 = jnp.float32

    beh = beh_ref[...]
    r = rid_ref[...]
    bidx = lax.broadcasted_iota(jnp.int32, (TB, B), 1)
    P = (r == bidx).astype(f32)

    ii = lax.broadcasted_iota(jnp.int32, (D, DD), 0)
    cc = lax.broadcasted_iota(jnp.int32, (D, DD), 1)
    r_div = (cc // D == ii).astype(f32)
    r_mod = (cc % D == ii).astype(f32)

    w1c = w1_ref[0:D, :]
    w1b = w1_ref[D:2 * D, :]
    w1o = w1_ref[2 * D:, :]

    cand = cand_ref[...]
    cand_tiled = jnp.dot(cand, r_mod, preferred_element_type=f32)
    c1b = jnp.dot(cand, w1c, preferred_element_type=f32) + b1_ref[...]

    xb = jnp.dot(beh, r_div, preferred_element_type=f32)
    xc = jnp.dot(P, cand_tiled, preferred_element_type=f32)
    outer = xb * xc
    h = (jnp.dot(outer, w1o, preferred_element_type=f32)
         + jnp.dot(beh, w1b, preferred_element_type=f32)
         + jnp.dot(P, c1b, preferred_element_type=f32))

    v_mean = jnp.full((H, 1), 1.0 / H, f32)
    mean = jnp.dot(h, v_mean, preferred_element_type=f32)
    msq = jnp.dot(h * h, v_mean, preferred_element_type=f32)
    var = msq - mean * mean + _EPS
    std = jnp.sqrt(var)
    p = jax.nn.sigmoid((h - mean) / (std + _EPS))
    hd = alpha_ref[...] * (1.0 - p) * h + p * h

    w = jnp.dot(hd, w2_ref[...], preferred_element_type=f32) + b2_ref[...]
    weighted = beh * w
    partial = lax.dot_general(P, weighted, (((0,), (0,)), ((), ())),
                              preferred_element_type=f32)

    @pl.when(i == 0)
    def _init():
        out_ref[...] = jnp.zeros_like(out_ref)

    out_ref[...] += partial


def kernel(candidate_tensor, behavior_flat_values, behavior_value_rowids, W1,
           b1, alpha, W2, b2):
    T, D = behavior_flat_values.shape
    B = candidate_tensor.shape[0]
    H = W1.shape[1]
    TB = 8192
    grid = T // TB

    rowids2 = behavior_value_rowids.reshape(T, 1)
    b1r = b1.reshape(1, H)
    alphar = alpha.reshape(1, H)
    b2r = b2.reshape(1, 1)

    return pl.pallas_call(
        _body,
        grid=(grid,),
        in_specs=[
            pl.BlockSpec((B, D), lambda i: (0, 0)),
            pl.BlockSpec((TB, D), lambda i: (i, 0)),
            pl.BlockSpec((TB, 1), lambda i: (i, 0)),
            pl.BlockSpec((D + D + D * D, H), lambda i: (0, 0)),
            pl.BlockSpec((1, H), lambda i: (0, 0)),
            pl.BlockSpec((1, H), lambda i: (0, 0)),
            pl.BlockSpec((H, 1), lambda i: (0, 0)),
            pl.BlockSpec((1, 1), lambda i: (0, 0)),
        ],
        out_specs=pl.BlockSpec((B, D), lambda i: (0, 0)),
        out_shape=jax.ShapeDtypeStruct((B, D), jnp.float32),
    )(candidate_tensor, behavior_flat_values, rowids2, W1, b1r, alphar, W2,
      b2r)


# speedup vs baseline: 1.0668x; 1.0668x over previous
import jax
import jax.numpy as jnp
from jax import lax
from jax.experimental import pallas as pl

_EPS = 1e-10


def _body(cand_ref, beh_ref, rid_ref, w1_ref, b1_ref, alpha_ref, w2_ref,
          b2_ref, out_ref):
    i = pl.program_id(0)
    TB, D = beh_ref.shape
    B = cand_ref.shape[0]
    H = w1_ref.shape[1]
    DD = D * D
    f32 = jnp.float32

    beh = beh_ref[...]
    r = rid_ref[...]
    bidx = lax.broadcasted_iota(jnp.int32, (TB, B), 1)
    P = (r == bidx).astype(f32)

    ii = lax.broadcasted_iota(jnp.int32, (D, DD), 0)
    cc = lax.broadcasted_iota(jnp.int32, (D, DD), 1)
    r_div = (cc // D == ii).astype(f32)
    r_mod = (cc % D == ii).astype(f32)

    w1c = w1_ref[0:D, :]
    w1b = w1_ref[D:2 * D, :]
    w1o = w1_ref[2 * D:, :]

    cand = cand_ref[...]
    cand_tiled = jnp.dot(cand, r_mod, preferred_element_type=f32)
    c1b = jnp.dot(cand, w1c, preferred_element_type=f32) + b1_ref[...]

    xb = jnp.dot(beh, r_div, preferred_element_type=f32)
    xc = jnp.dot(P, cand_tiled, preferred_element_type=f32)
    outer = xb * xc
    h = (jnp.dot(outer, w1o, preferred_element_type=f32)
         + jnp.dot(beh, w1b, preferred_element_type=f32)
         + jnp.dot(P, c1b, preferred_element_type=f32))

    v_mean = jnp.full((H, H), 1.0 / H, f32)
    mean = jnp.dot(h, v_mean, preferred_element_type=f32)
    msq = jnp.dot(h * h, v_mean, preferred_element_type=f32)
    var = msq - mean * mean + _EPS
    std = jnp.sqrt(var)
    p = jax.nn.sigmoid((h - mean) / (std + _EPS))
    alpha_v = alpha_ref[...]
    hd = h * (alpha_v + (1.0 - alpha_v) * p)

    w = jnp.dot(hd, w2_ref[...], preferred_element_type=f32) + b2_ref[...]
    weighted = beh * w
    partial = lax.dot_general(P, weighted, (((0,), (0,)), ((), ())),
                              preferred_element_type=f32)

    @pl.when(i == 0)
    def _init():
        out_ref[...] = jnp.zeros_like(out_ref)

    out_ref[...] += partial


def kernel(candidate_tensor, behavior_flat_values, behavior_value_rowids, W1,
           b1, alpha, W2, b2):
    T, D = behavior_flat_values.shape
    B = candidate_tensor.shape[0]
    H = W1.shape[1]
    TB = 4096
    grid = T // TB

    rowids2 = behavior_value_rowids.reshape(T, 1)
    b1r = b1.reshape(1, H)
    alphar = alpha.reshape(1, H)
    w2rep = jnp.tile(W2.reshape(H, 1), (1, D))
    b2r = b2.reshape(1, 1)

    return pl.pallas_call(
        _body,
        grid=(grid,),
        in_specs=[
            pl.BlockSpec((B, D), lambda i: (0, 0)),
            pl.BlockSpec((TB, D), lambda i: (i, 0)),
            pl.BlockSpec((TB, 1), lambda i: (i, 0)),
            pl.BlockSpec((D + D + D * D, H), lambda i: (0, 0)),
            pl.BlockSpec((1, H), lambda i: (0, 0)),
            pl.BlockSpec((1, H), lambda i: (0, 0)),
            pl.BlockSpec((H, D), lambda i: (0, 0)),
            pl.BlockSpec((1, 1), lambda i: (0, 0)),
        ],
        out_specs=pl.BlockSpec((B, D), lambda i: (0, 0)),
        out_shape=jax.ShapeDtypeStruct((B, D), jnp.float32),
    )(candidate_tensor, behavior_flat_values, rowids2, W1, b1r, alphar, w2rep,
      b2r)


# speedup vs baseline: 1.1225x; 1.0523x over previous
import jax
import jax.numpy as jnp
from jax import lax
from jax.experimental import pallas as pl

_EPS = 1e-10


def _body(cand_ref, beh_ref, rid_ref, w1_ref, b1_ref, alpha_ref, w2_ref,
          b2_ref, rdiv_ref, rmod_ref, out_ref):
    i = pl.program_id(0)
    TB, D = beh_ref.shape
    B = cand_ref.shape[0]
    H = w1_ref.shape[1]
    f32 = jnp.float32

    beh = beh_ref[...]
    r = rid_ref[...]
    bidx = lax.broadcasted_iota(jnp.int32, (TB, B), 1)
    P = (r == bidx).astype(f32)

    w1c = w1_ref[0:D, :]
    w1b = w1_ref[D:2 * D, :]
    w1o = w1_ref[2 * D:, :]

    cand = cand_ref[...]
    cand_tiled = jnp.dot(cand, rmod_ref[...], preferred_element_type=f32)
    c1b = jnp.dot(cand, w1c, preferred_element_type=f32) + b1_ref[...]

    xb = jnp.dot(beh, rdiv_ref[...], preferred_element_type=f32)
    xc = jnp.dot(P, cand_tiled, preferred_element_type=f32)
    outer = xb * xc
    h = (jnp.dot(outer, w1o, preferred_element_type=f32)
         + jnp.dot(beh, w1b, preferred_element_type=f32)
         + jnp.dot(P, c1b, preferred_element_type=f32))

    v_mean = jnp.full((H, H), 1.0 / H, f32)
    mean = jnp.dot(h, v_mean, preferred_element_type=f32)
    msq = jnp.dot(h * h, v_mean, preferred_element_type=f32)
    var = jnp.maximum(msq - mean * mean, 0.0) + _EPS
    inv2 = 0.5 * lax.rsqrt(var)
    th = jnp.tanh((h - mean) * inv2)
    alpha_v = alpha_ref[...]
    c0 = 0.5 * (1.0 + alpha_v)
    c1 = 0.5 * (1.0 - alpha_v)
    hd = h * (c0 + c1 * th)

    w = jnp.dot(hd, w2_ref[...], preferred_element_type=f32) + b2_ref[...]
    weighted = beh * w
    partial = lax.dot_general(P, weighted, (((0,), (0,)), ((), ())),
                              preferred_element_type=f32)

    @pl.when(i == 0)
    def _init():
        out_ref[...] = jnp.zeros_like(out_ref)

    out_ref[...] += partial


def kernel(candidate_tensor, behavior_flat_values, behavior_value_rowids, W1,
           b1, alpha, W2, b2):
    T, D = behavior_flat_values.shape
    B = candidate_tensor.shape[0]
    H = W1.shape[1]
    DD = D * D
    TB = 4096
    grid = T // TB

    rowids2 = behavior_value_rowids.reshape(T, 1)
    b1r = b1.reshape(1, H)
    alphar = alpha.reshape(1, H)
    w2rep = jnp.tile(W2.reshape(H, 1), (1, D))
    b2r = b2.reshape(1, 1)
    cols = jnp.arange(DD, dtype=jnp.int32)
    rows = jnp.arange(D, dtype=jnp.int32)
    r_div = (cols[None, :] // D == rows[:, None]).astype(jnp.float32)
    r_mod = (cols[None, :] % D == rows[:, None]).astype(jnp.float32)

    return pl.pallas_call(
        _body,
        grid=(grid,),
        in_specs=[
            pl.BlockSpec((B, D), lambda i: (0, 0)),
            pl.BlockSpec((TB, D), lambda i: (i, 0)),
            pl.BlockSpec((TB, 1), lambda i: (i, 0)),
            pl.BlockSpec((D + D + DD, H), lambda i: (0, 0)),
            pl.BlockSpec((1, H), lambda i: (0, 0)),
            pl.BlockSpec((1, H), lambda i: (0, 0)),
            pl.BlockSpec((H, D), lambda i: (0, 0)),
            pl.BlockSpec((1, 1), lambda i: (0, 0)),
            pl.BlockSpec((D, DD), lambda i: (0, 0)),
            pl.BlockSpec((D, DD), lambda i: (0, 0)),
        ],
        out_specs=pl.BlockSpec((B, D), lambda i: (0, 0)),
        out_shape=jax.ShapeDtypeStruct((B, D), jnp.float32),
    )(candidate_tensor, behavior_flat_values, rowids2, W1, b1r, alphar, w2rep,
      b2r, r_div, r_mod)
